# Initial kernel scaffold; baseline (speedup 1.0000x reference)
import jax
import jax.numpy as jnp
from jax.experimental import pallas as pl


def kernel(x, adj, W1, b1, W2, b2):
    raise NotImplementedError("write your pallas kernel here")


# speedup vs baseline: 3.5586x; 3.5586x over previous
import functools

import jax
import jax.numpy as jnp
from jax import lax
from jax.experimental import pallas as pl
from jax.experimental.pallas import tpu as pltpu
from jax.experimental.pallas import tpu_sc as plsc

N_NODES = 10000
N_EDGES = 160000
NFEAT = 256
NHID = 256
NCLASS = 64

NC = 2
NS = 16
LANES = 16

ROW_BLK = 1000
N_ROW_BLK = N_NODES // ROW_BLK

DRCH = 200
NDRCH = N_NODES // DRCH
DR_PER_TILE = -(-NDRCH // NS)


@functools.lru_cache(maxsize=None)
def _mesh():
    return plsc.VectorSubcoreMesh(
        core_axis_name="c", subcore_axis_name="s", num_cores=NC, num_subcores=NS
    )


def _mm1_body(x_ref, w_ref, o0_ref, o1_ref):
    s = jnp.dot(x_ref[...], w_ref[...], preferred_element_type=jnp.float32)
    o0_ref[...] = s[:, :128]
    o1_ref[...] = s[:, 128:]


def _mm1(x, W1):
    return pl.pallas_call(
        _mm1_body,
        grid=(N_ROW_BLK,),
        in_specs=[
            pl.BlockSpec((ROW_BLK, NFEAT), lambda i: (i, 0)),
            pl.BlockSpec((NFEAT, NHID), lambda i: (0, 0)),
        ],
        out_specs=[
            pl.BlockSpec((ROW_BLK, 128), lambda i: (i, 0)),
            pl.BlockSpec((ROW_BLK, 128), lambda i: (i, 0)),
        ],
        out_shape=[
            jax.ShapeDtypeStruct((N_NODES, 128), jnp.float32),
            jax.ShapeDtypeStruct((N_NODES, 128), jnp.float32),
        ],
    )(x, W1)


EPT1 = N_EDGES // NS
CH1 = 80
NCH1 = EPT1 // CH1


@functools.lru_cache(maxsize=None)
def _spmm1_kernel():
    @functools.partial(
        pl.kernel,
        out_type=[
            jax.ShapeDtypeStruct((N_NODES, 128), jnp.float32),
            jax.ShapeDtypeStruct((N_NODES, 128), jnp.float32),
        ],
        mesh=_mesh(),
        scratch_types=[
            pltpu.VMEM((CH1,), jnp.int32),
            pltpu.VMEM((CH1,), jnp.int32),
            pltpu.VMEM((CH1, 128), jnp.float32),
            pltpu.VMEM((DRCH, 128), jnp.float32),
            pltpu.VMEM((128,), jnp.float32),
            pltpu.VMEM_SHARED((N_NODES, 128), jnp.float32),
            pltpu.SemaphoreType.DMA,
        ],
    )
    def spmm1(s0_hbm, s1_hbm, src_hbm, dst_hbm, b1a_hbm, b1b_hbm, zeros_hbm,
              h0_hbm, h1_hbm,
              src_v, dst_v, rows_v, buf_v, bias_v, acc, sem):
        c = lax.axis_index("c")
        s = lax.axis_index("s")

        pltpu.sync_copy(zeros_hbm, buf_v)
        for i in range(DR_PER_TILE):
            ch = s + NS * i

            @pl.when(ch < NDRCH)
            def _():
                pltpu.sync_copy(buf_v, acc.at[pl.ds(ch * DRCH, DRCH)])

        @pl.when(c == 0)
        def _():
            pltpu.sync_copy(b1a_hbm, bias_v)

        @pl.when(c == 1)
        def _():
            pltpu.sync_copy(b1b_hbm, bias_v)

        plsc.subcore_barrier()

        ebase = s * EPT1

        def chunk_body(j, carry):
            off = ebase + j * CH1
            pltpu.sync_copy(src_hbm.at[pl.ds(off, CH1)], src_v)
            pltpu.sync_copy(dst_hbm.at[pl.ds(off, CH1)], dst_v)

            @pl.when(c == 0)
            def _():
                pltpu.async_copy(s0_hbm.at[src_v], rows_v, sem).wait()

            @pl.when(c == 1)
            def _():
                pltpu.async_copy(s1_hbm.at[src_v], rows_v, sem).wait()

            pltpu.sync_copy(rows_v, acc.at[dst_v], add=True)
            return carry

        lax.fori_loop(0, NCH1, chunk_body, 0)
        plsc.subcore_barrier()

        for i in range(DR_PER_TILE):
            ch = s + NS * i

            @pl.when(ch < NDRCH)
            def _():
                r0 = ch * DRCH
                pltpu.sync_copy(acc.at[pl.ds(r0, DRCH)], buf_v)

                def row_body(r, carry2):
                    for k in range(128 // LANES):
                        v = buf_v[r, pl.ds(k * LANES, LANES)]
                        v = v + bias_v[pl.ds(k * LANES, LANES)]
                        buf_v[r, pl.ds(k * LANES, LANES)] = jnp.maximum(v, 0.0)
                    return carry2

                lax.fori_loop(0, DRCH, row_body, 0)

                @pl.when(c == 0)
                def _():
                    pltpu.sync_copy(buf_v, h0_hbm.at[pl.ds(r0, DRCH)])

                @pl.when(c == 1)
                def _():
                    pltpu.sync_copy(buf_v, h1_hbm.at[pl.ds(r0, DRCH)])

    return spmm1


NCLS_P = 128


def _mm2_body(h0_ref, h1_ref, w2_ref, o_ref):
    a = jnp.dot(h0_ref[...], w2_ref[:128, :], preferred_element_type=jnp.float32)
    b = jnp.dot(h1_ref[...], w2_ref[128:, :], preferred_element_type=jnp.float32)
    o_ref[...] = a + b


def _mm2(h0, h1, W2):
    return pl.pallas_call(
        _mm2_body,
        grid=(N_ROW_BLK,),
        in_specs=[
            pl.BlockSpec((ROW_BLK, 128), lambda i: (i, 0)),
            pl.BlockSpec((ROW_BLK, 128), lambda i: (i, 0)),
            pl.BlockSpec((NHID, NCLS_P), lambda i: (0, 0)),
        ],
        out_specs=pl.BlockSpec((ROW_BLK, NCLS_P), lambda i: (i, 0)),
        out_shape=jax.ShapeDtypeStruct((N_NODES, NCLS_P), jnp.float32),
    )(h0, h1, W2)


EPT2 = N_EDGES // (NC * NS)
CH2 = 80
NCH2 = EPT2 // CH2
REM2 = EPT2 - NCH2 * CH2


@functools.lru_cache(maxsize=None)
def _spmm2_kernel():
    @functools.partial(
        pl.kernel,
        out_type=[
            jax.ShapeDtypeStruct((N_NODES, NCLS_P), jnp.float32),
            jax.ShapeDtypeStruct((N_NODES, NCLS_P), jnp.float32),
        ],
        mesh=_mesh(),
        scratch_types=[
            pltpu.VMEM((CH2,), jnp.int32),
            pltpu.VMEM((CH2,), jnp.int32),
            pltpu.VMEM((CH2, NCLS_P), jnp.float32),
            pltpu.VMEM((REM2,), jnp.int32),
            pltpu.VMEM((REM2,), jnp.int32),
            pltpu.VMEM((REM2, NCLS_P), jnp.float32),
            pltpu.VMEM((DRCH, NCLS_P), jnp.float32),
            pltpu.VMEM_SHARED((N_NODES, NCLS_P), jnp.float32),
            pltpu.SemaphoreType.DMA,
        ],
    )
    def spmm2(s2_hbm, src_hbm, dst_hbm, zeros_hbm,
              p0_hbm, p1_hbm,
              src_v, dst_v, rows_v, srcr_v, dstr_v, rowsr_v, buf_v, acc, sem):
        c = lax.axis_index("c")
        s = lax.axis_index("s")
        wid = c * NS + s

        pltpu.sync_copy(zeros_hbm, buf_v)
        for i in range(DR_PER_TILE):
            ch = s + NS * i

            @pl.when(ch < NDRCH)
            def _():
                pltpu.sync_copy(buf_v, acc.at[pl.ds(ch * DRCH, DRCH)])

        plsc.subcore_barrier()

        ebase = wid * EPT2

        def chunk_body(j, carry):
            off = ebase + j * CH2
            pltpu.sync_copy(src_hbm.at[pl.ds(off, CH2)], src_v)
            pltpu.sync_copy(dst_hbm.at[pl.ds(off, CH2)], dst_v)
            pltpu.async_copy(s2_hbm.at[src_v], rows_v, sem).wait()
            pltpu.sync_copy(rows_v, acc.at[dst_v], add=True)
            return carry

        lax.fori_loop(0, NCH2, chunk_body, 0)

        offr = ebase + NCH2 * CH2
        pltpu.sync_copy(src_hbm.at[pl.ds(offr, REM2)], srcr_v)
        pltpu.sync_copy(dst_hbm.at[pl.ds(offr, REM2)], dstr_v)
        pltpu.async_copy(s2_hbm.at[srcr_v], rowsr_v, sem).wait()
        pltpu.sync_copy(rowsr_v, acc.at[dstr_v], add=True)

        plsc.subcore_barrier()

        for i in range(DR_PER_TILE):
            ch = s + NS * i

            @pl.when(ch < NDRCH)
            def _():
                r0 = ch * DRCH
                pltpu.sync_copy(acc.at[pl.ds(r0, DRCH)], buf_v)

                @pl.when(c == 0)
                def _():
                    pltpu.sync_copy(buf_v, p0_hbm.at[pl.ds(r0, DRCH)])

                @pl.when(c == 1)
                def _():
                    pltpu.sync_copy(buf_v, p1_hbm.at[pl.ds(r0, DRCH)])

    return spmm2


def _comb_body(p0_ref, p1_ref, b2_ref, o_ref):
    o_ref[...] = p0_ref[:, :NCLASS] + p1_ref[:, :NCLASS] + b2_ref[0:1, :]


def _comb(p0, p1, b2):
    return pl.pallas_call(
        _comb_body,
        grid=(N_ROW_BLK,),
        in_specs=[
            pl.BlockSpec((ROW_BLK, NCLS_P), lambda i: (i, 0)),
            pl.BlockSpec((ROW_BLK, NCLS_P), lambda i: (i, 0)),
            pl.BlockSpec((8, NCLASS), lambda i: (0, 0)),
        ],
        out_specs=pl.BlockSpec((ROW_BLK, NCLASS), lambda i: (i, 0)),
        out_shape=jax.ShapeDtypeStruct((N_NODES, NCLASS), jnp.float32),
    )(p0, p1, b2)


@jax.jit
def kernel(x, adj, W1, b1, W2, b2):
    src = adj[0].astype(jnp.int32)
    dst = adj[1].astype(jnp.int32)
    zeros1 = jnp.zeros((DRCH, 128), jnp.float32)
    zeros2 = jnp.zeros((DRCH, NCLS_P), jnp.float32)

    s0, s1 = _mm1(x, W1)
    h0, h1 = _spmm1_kernel()(s0, s1, src, dst, b1[:128], b1[128:], zeros1)
    W2p = jnp.pad(W2, ((0, 0), (0, NCLS_P - NCLASS)))
    s2 = _mm2(h0, h1, W2p)
    p0, p1 = _spmm2_kernel()(s2, src, dst, zeros2)
    b2r = jnp.broadcast_to(b2.reshape(1, NCLASS), (8, NCLASS))
    return _comb(p0, p1, b2r)


# speedup vs baseline: 6.7458x; 1.8956x over previous
import functools

import jax
import jax.numpy as jnp
from jax import lax
from jax.experimental import pallas as pl
from jax.experimental.pallas import tpu as pltpu
from jax.experimental.pallas import tpu_sc as plsc

N_NODES = 10000
N_EDGES = 160000
NFEAT = 256
NHID = 256
NCLASS = 64
NCLS_P = 128

NC = 2
NS = 16

ROW_BLK = 1000
N_ROW_BLK = N_NODES // ROW_BLK

DRCH = 80
NDRCH = N_NODES // DRCH
DR_PER_TILE = -(-NDRCH // NS)

CHW = 100
IB = 25
NB1 = N_EDGES // NS // (IB * CHW)
NB2 = N_EDGES // (NC * NS) // (IB * CHW)


@functools.lru_cache(maxsize=None)
def _mesh():
    return plsc.VectorSubcoreMesh(
        core_axis_name="c", subcore_axis_name="s", num_cores=NC, num_subcores=NS
    )


def _mm1_body(x_ref, w_ref, o_ref):
    o_ref[...] = jnp.dot(x_ref[...], w_ref[...],
                         preferred_element_type=jnp.float32)


def _mm1(x, W1):
    return pl.pallas_call(
        _mm1_body,
        grid=(NC, N_ROW_BLK),
        in_specs=[
            pl.BlockSpec((ROW_BLK, NFEAT), lambda h, i: (i, 0)),
            pl.BlockSpec((NFEAT, 128), lambda h, i: (0, h)),
        ],
        out_specs=pl.BlockSpec((ROW_BLK, 128),
                               lambda h, i: (h * N_ROW_BLK + i, 0)),
        out_shape=jax.ShapeDtypeStruct((NC * N_NODES, 128), jnp.float32),
    )(x, W1)


def _zero_acc(zeros_hbm, buf_v, acc, s):
    pltpu.sync_copy(zeros_hbm, buf_v)
    for i in range(DR_PER_TILE):
        ch = s + NS * i

        @pl.when(ch < NDRCH)
        def _():
            pltpu.sync_copy(buf_v, acc.at[pl.ds(ch * DRCH, DRCH)])


@functools.lru_cache(maxsize=None)
def _spmm1_kernel():
    @functools.partial(
        pl.kernel,
        out_type=jax.ShapeDtypeStruct((NC * N_NODES, 128), jnp.float32),
        mesh=_mesh(),
        scratch_types=[
            pltpu.VMEM((IB, CHW), jnp.int32),
            pltpu.VMEM((IB, CHW), jnp.int32),
            pltpu.VMEM((CHW, 128), jnp.float32),
            pltpu.VMEM((CHW, 128), jnp.float32),
            pltpu.VMEM((DRCH, 128), jnp.float32),
            pltpu.VMEM_SHARED((N_NODES, 128), jnp.float32),
            pltpu.SemaphoreType.DMA,
        ],
    )
    def spmm1(sup_hbm, src_hbm, dst_hbm, zeros_hbm, h_hbm,
              isrc_v, idst_v, rows0_v, rows1_v, buf_v, acc, sem):
        c = lax.axis_index("c")
        s = lax.axis_index("s")

        _zero_acc(zeros_hbm, buf_v, acc, s)
        plsc.subcore_barrier()

        def gather(j, buf):
            pltpu.async_copy(sup_hbm.at[isrc_v.at[j]], buf, sem)

        def wait_gather(buf):
            pltpu.make_async_copy(sup_hbm.at[isrc_v.at[0]], buf, sem).wait()

        def block_body(b, carry):
            pltpu.sync_copy(src_hbm.at[c, s, b], isrc_v)
            pltpu.sync_copy(dst_hbm.at[s, b], idst_v)
            bufs = (rows0_v, rows1_v)
            gather(0, bufs[0])
            for j in range(IB):
                wait_gather(bufs[j % 2])
                if j + 1 < IB:
                    gather(j + 1, bufs[(j + 1) % 2])
                pltpu.sync_copy(bufs[j % 2], acc.at[idst_v.at[j]], add=True)
            return carry

        lax.fori_loop(0, NB1, block_body, 0)

        plsc.subcore_barrier()

        for i in range(DR_PER_TILE):
            ch = s + NS * i

            @pl.when(ch < NDRCH)
            def _():
                r0 = ch * DRCH
                pltpu.sync_copy(acc.at[pl.ds(r0, DRCH)], buf_v)
                pltpu.sync_copy(buf_v, h_hbm.at[pl.ds(c * N_NODES + r0, DRCH)])

    return spmm1


def _mm2_body(h0_ref, h1_ref, b1a_ref, b1b_ref, w2_ref, o_ref):
    h0 = jnp.maximum(h0_ref[...] + b1a_ref[0:1, :], 0.0)
    h1 = jnp.maximum(h1_ref[...] + b1b_ref[0:1, :], 0.0)
    a = jnp.dot(h0, w2_ref[:128, :], preferred_element_type=jnp.float32)
    b = jnp.dot(h1, w2_ref[128:, :], preferred_element_type=jnp.float32)
    o_ref[...] = a + b


def _mm2(h_all, b1a, b1b, W2):
    return pl.pallas_call(
        _mm2_body,
        grid=(N_ROW_BLK,),
        in_specs=[
            pl.BlockSpec((ROW_BLK, 128), lambda i: (i, 0)),
            pl.BlockSpec((ROW_BLK, 128), lambda i: (N_ROW_BLK + i, 0)),
            pl.BlockSpec((8, 128), lambda i: (0, 0)),
            pl.BlockSpec((8, 128), lambda i: (0, 0)),
            pl.BlockSpec((NHID, NCLS_P), lambda i: (0, 0)),
        ],
        out_specs=pl.BlockSpec((ROW_BLK, NCLS_P), lambda i: (i, 0)),
        out_shape=jax.ShapeDtypeStruct((N_NODES, NCLS_P), jnp.float32),
    )(h_all, h_all, b1a, b1b, W2)


@functools.lru_cache(maxsize=None)
def _spmm2_kernel():
    @functools.partial(
        pl.kernel,
        out_type=jax.ShapeDtypeStruct((NC * N_NODES, NCLS_P), jnp.float32),
        mesh=_mesh(),
        scratch_types=[
            pltpu.VMEM((IB, CHW), jnp.int32),
            pltpu.VMEM((IB, CHW), jnp.int32),
            pltpu.VMEM((CHW, NCLS_P), jnp.float32),
            pltpu.VMEM((CHW, NCLS_P), jnp.float32),
            pltpu.VMEM((DRCH, NCLS_P), jnp.float32),
            pltpu.VMEM_SHARED((N_NODES, NCLS_P), jnp.float32),
            pltpu.SemaphoreType.DMA,
        ],
    )
    def spmm2(s2_hbm, src_hbm, dst_hbm, zeros_hbm, p_hbm,
              isrc_v, idst_v, rows0_v, rows1_v, buf_v, acc, sem):
        c = lax.axis_index("c")
        s = lax.axis_index("s")
        wid = c * NS + s

        _zero_acc(zeros_hbm, buf_v, acc, s)
        plsc.subcore_barrier()

        def gather(j, buf):
            pltpu.async_copy(s2_hbm.at[isrc_v.at[j]], buf, sem)

        def wait_gather(buf):
            pltpu.make_async_copy(s2_hbm.at[isrc_v.at[0]], buf, sem).wait()

        def block_body(b, carry):
            pltpu.sync_copy(src_hbm.at[wid, b], isrc_v)
            pltpu.sync_copy(dst_hbm.at[wid, b], idst_v)
            bufs = (rows0_v, rows1_v)
            gather(0, bufs[0])
            for j in range(IB):
                wait_gather(bufs[j % 2])
                if j + 1 < IB:
                    gather(j + 1, bufs[(j + 1) % 2])
                pltpu.sync_copy(bufs[j % 2], acc.at[idst_v.at[j]], add=True)
            return carry

        lax.fori_loop(0, NB2, block_body, 0)

        plsc.subcore_barrier()

        for i in range(DR_PER_TILE):
            ch = s + NS * i

            @pl.when(ch < NDRCH)
            def _():
                r0 = ch * DRCH
                pltpu.sync_copy(acc.at[pl.ds(r0, DRCH)], buf_v)
                pltpu.sync_copy(buf_v, p_hbm.at[pl.ds(c * N_NODES + r0, DRCH)])

    return spmm2


def _comb_body(p0_ref, p1_ref, b2_ref, o_ref):
    o_ref[...] = p0_ref[:, :NCLASS] + p1_ref[:, :NCLASS] + b2_ref[0:1, :]


def _comb(p_all, b2):
    return pl.pallas_call(
        _comb_body,
        grid=(N_ROW_BLK,),
        in_specs=[
            pl.BlockSpec((ROW_BLK, NCLS_P), lambda i: (i, 0)),
            pl.BlockSpec((ROW_BLK, NCLS_P), lambda i: (N_ROW_BLK + i, 0)),
            pl.BlockSpec((8, NCLASS), lambda i: (0, 0)),
        ],
        out_specs=pl.BlockSpec((ROW_BLK, NCLASS), lambda i: (i, 0)),
        out_shape=jax.ShapeDtypeStruct((N_NODES, NCLASS), jnp.float32),
    )(p_all, p_all, b2)


@jax.jit
def kernel(x, adj, W1, b1, W2, b2):
    src = adj[0].astype(jnp.int32)
    dst = adj[1].astype(jnp.int32)
    src1 = jnp.stack([src, src + N_NODES]).reshape(NC, NS, NB1, IB, CHW)
    dst1 = dst.reshape(NS, NB1, IB, CHW)
    src2 = src.reshape(NC * NS, NB2, IB, CHW)
    dst2 = dst.reshape(NC * NS, NB2, IB, CHW)
    zeros1 = jnp.zeros((DRCH, 128), jnp.float32)
    zeros2 = jnp.zeros((DRCH, NCLS_P), jnp.float32)
    b1a = jnp.broadcast_to(b1[:128].reshape(1, 128), (8, 128))
    b1b = jnp.broadcast_to(b1[128:].reshape(1, 128), (8, 128))
    b2r = jnp.broadcast_to(b2.reshape(1, NCLASS), (8, NCLASS))
    W2p = jnp.pad(W2, ((0, 0), (0, NCLS_P - NCLASS)))

    sup = _mm1(x, W1)
    h_all = _spmm1_kernel()(sup, src1, dst1, zeros1)
    s2 = _mm2(h_all, b1a, b1b, W2p)
    p_all = _spmm2_kernel()(s2, src2, dst2, zeros2)
    return _comb(p_all, b2r)


# speedup vs baseline: 8.4720x; 1.2559x over previous
import functools

import jax
import jax.numpy as jnp
from jax import lax
from jax.experimental import pallas as pl
from jax.experimental.pallas import tpu as pltpu
from jax.experimental.pallas import tpu_sc as plsc

N_NODES = 10000
N_EDGES = 160000
NFEAT = 256
NHID = 256
NCLASS = 64
NCLS_P = 128

NC = 2
NS = 16

ROW_BLK = 1000
N_ROW_BLK = N_NODES // ROW_BLK

DRCH = 80
NDRCH = N_NODES // DRCH
DR_PER_TILE = -(-NDRCH // NS)

CHW1 = 80
CHW2 = 100
IB = 25
NB1 = N_EDGES // NS // (IB * CHW1)
NB2 = N_EDGES // (NC * NS) // (IB * CHW2)
D1 = 4
D2 = 3


@functools.lru_cache(maxsize=None)
def _mesh():
    return plsc.VectorSubcoreMesh(
        core_axis_name="c", subcore_axis_name="s", num_cores=NC, num_subcores=NS
    )


def _mm1_body(x_ref, w_ref, o_ref):
    o_ref[...] = jnp.dot(x_ref[...], w_ref[...],
                         preferred_element_type=jnp.float32)


def _mm1(x, W1):
    return pl.pallas_call(
        _mm1_body,
        grid=(NC, N_ROW_BLK),
        in_specs=[
            pl.BlockSpec((ROW_BLK, NFEAT), lambda h, i: (i, 0)),
            pl.BlockSpec((NFEAT, 128), lambda h, i: (0, h)),
        ],
        out_specs=pl.BlockSpec((ROW_BLK, 128),
                               lambda h, i: (h * N_ROW_BLK + i, 0)),
        out_shape=jax.ShapeDtypeStruct((NC * N_NODES, 128), jnp.float32),
    )(x, W1)


def _zero_acc(zeros_hbm, buf_v, acc, s):
    pltpu.sync_copy(zeros_hbm, buf_v)
    for i in range(DR_PER_TILE):
        ch = s + NS * i

        @pl.when(ch < NDRCH)
        def _():
            pltpu.sync_copy(buf_v, acc.at[pl.ds(ch * DRCH, DRCH)])


@functools.lru_cache(maxsize=None)
def _spmm1_kernel():
    @functools.partial(
        pl.kernel,
        out_type=jax.ShapeDtypeStruct((NC * N_NODES, 128), jnp.float32),
        mesh=_mesh(),
        scratch_types=[
            pltpu.VMEM((IB, CHW1), jnp.int32),
            pltpu.VMEM((IB, CHW1), jnp.int32),
            [pltpu.VMEM((CHW1, 128), jnp.float32) for _ in range(D1)],
            pltpu.VMEM_SHARED((N_NODES, 128), jnp.float32),
            [pltpu.SemaphoreType.DMA for _ in range(D1)],
        ],
    )
    def spmm1(sup_hbm, src_hbm, dst_hbm, zeros_hbm, h_hbm,
              isrc_v, idst_v, bufs, acc, sems):
        c = lax.axis_index("c")
        s = lax.axis_index("s")

        _zero_acc(zeros_hbm, bufs[0], acc, s)
        plsc.subcore_barrier()

        def gather(j, k):
            pltpu.async_copy(sup_hbm.at[isrc_v.at[j]], bufs[k], sems[k])

        def wait_gather(k):
            pltpu.make_async_copy(
                sup_hbm.at[isrc_v.at[0]], bufs[k], sems[k]).wait()

        def block_body(b, carry):
            pltpu.sync_copy(src_hbm.at[c, s, b], isrc_v)
            pltpu.sync_copy(dst_hbm.at[s, b], idst_v)
            for k in range(D1 - 1):
                gather(k, k)
            for j in range(IB):
                wait_gather(j % D1)
                if j + D1 - 1 < IB:
                    gather(j + D1 - 1, (j + D1 - 1) % D1)
                pltpu.sync_copy(bufs[j % D1], acc.at[idst_v.at[j]], add=True)
            return carry

        lax.fori_loop(0, NB1, block_body, 0)

        plsc.subcore_barrier()

        for i in range(DR_PER_TILE):
            ch = s + NS * i

            @pl.when(ch < NDRCH)
            def _():
                r0 = ch * DRCH
                pltpu.sync_copy(acc.at[pl.ds(r0, DRCH)], bufs[0])
                pltpu.sync_copy(bufs[0],
                                h_hbm.at[pl.ds(c * N_NODES + r0, DRCH)])

    return spmm1


def _mm2_body(h0_ref, h1_ref, b1a_ref, b1b_ref, w2_ref, o_ref):
    h0 = jnp.maximum(h0_ref[...] + b1a_ref[0:1, :], 0.0)
    h1 = jnp.maximum(h1_ref[...] + b1b_ref[0:1, :], 0.0)
    a = jnp.dot(h0, w2_ref[:128, :], preferred_element_type=jnp.float32)
    b = jnp.dot(h1, w2_ref[128:, :], preferred_element_type=jnp.float32)
    o_ref[...] = a + b


def _mm2(h_all, b1a, b1b, W2):
    return pl.pallas_call(
        _mm2_body,
        grid=(N_ROW_BLK,),
        in_specs=[
            pl.BlockSpec((ROW_BLK, 128), lambda i: (i, 0)),
            pl.BlockSpec((ROW_BLK, 128), lambda i: (N_ROW_BLK + i, 0)),
            pl.BlockSpec((8, 128), lambda i: (0, 0)),
            pl.BlockSpec((8, 128), lambda i: (0, 0)),
            pl.BlockSpec((NHID, NCLS_P), lambda i: (0, 0)),
        ],
        out_specs=pl.BlockSpec((ROW_BLK, NCLS_P), lambda i: (i, 0)),
        out_shape=jax.ShapeDtypeStruct((N_NODES, NCLS_P), jnp.float32),
    )(h_all, h_all, b1a, b1b, W2)


@functools.lru_cache(maxsize=None)
def _spmm2_kernel():
    @functools.partial(
        pl.kernel,
        out_type=jax.ShapeDtypeStruct((NC * N_NODES, NCLS_P), jnp.float32),
        mesh=_mesh(),
        scratch_types=[
            pltpu.VMEM((IB, CHW2), jnp.int32),
            pltpu.VMEM((IB, CHW2), jnp.int32),
            [pltpu.VMEM((CHW2, NCLS_P), jnp.float32) for _ in range(D2)],
            pltpu.VMEM_SHARED((N_NODES, NCLS_P), jnp.float32),
            [pltpu.SemaphoreType.DMA for _ in range(D2)],
        ],
    )
    def spmm2(s2_hbm, src_hbm, dst_hbm, zeros_hbm, p_hbm,
              isrc_v, idst_v, bufs, acc, sems):
        c = lax.axis_index("c")
        s = lax.axis_index("s")
        wid = c * NS + s

        _zero_acc(zeros_hbm, bufs[0].at[pl.ds(0, DRCH)], acc, s)
        plsc.subcore_barrier()

        def gather(j, k):
            pltpu.async_copy(s2_hbm.at[isrc_v.at[j]], bufs[k], sems[k])

        def wait_gather(k):
            pltpu.make_async_copy(
                s2_hbm.at[isrc_v.at[0]], bufs[k], sems[k]).wait()

        def block_body(b, carry):
            pltpu.sync_copy(src_hbm.at[wid, b], isrc_v)
            pltpu.sync_copy(dst_hbm.at[wid, b], idst_v)
            for k in range(D2 - 1):
                gather(k, k)
            for j in range(IB):
                wait_gather(j % D2)
                if j + D2 - 1 < IB:
                    gather(j + D2 - 1, (j + D2 - 1) % D2)
                pltpu.sync_copy(bufs[j % D2], acc.at[idst_v.at[j]], add=True)
            return carry

        lax.fori_loop(0, NB2, block_body, 0)

        plsc.subcore_barrier()

        for i in range(DR_PER_TILE):
            ch = s + NS * i

            @pl.when(ch < NDRCH)
            def _():
                r0 = ch * DRCH
                pltpu.sync_copy(acc.at[pl.ds(r0, DRCH)], bufs[0].at[pl.ds(0, DRCH)])
                pltpu.sync_copy(bufs[0].at[pl.ds(0, DRCH)],
                                p_hbm.at[pl.ds(c * N_NODES + r0, DRCH)])

    return spmm2


def _comb_body(p0_ref, p1_ref, b2_ref, o_ref):
    o_ref[...] = p0_ref[:, :NCLASS] + p1_ref[:, :NCLASS] + b2_ref[0:1, :]


def _comb(p_all, b2):
    return pl.pallas_call(
        _comb_body,
        grid=(N_ROW_BLK,),
        in_specs=[
            pl.BlockSpec((ROW_BLK, NCLS_P), lambda i: (i, 0)),
            pl.BlockSpec((ROW_BLK, NCLS_P), lambda i: (N_ROW_BLK + i, 0)),
            pl.BlockSpec((8, NCLASS), lambda i: (0, 0)),
        ],
        out_specs=pl.BlockSpec((ROW_BLK, NCLASS), lambda i: (i, 0)),
        out_shape=jax.ShapeDtypeStruct((N_NODES, NCLASS), jnp.float32),
    )(p_all, p_all, b2)


@jax.jit
def kernel(x, adj, W1, b1, W2, b2):
    src = adj[0].astype(jnp.int32)
    dst = adj[1].astype(jnp.int32)
    src1 = jnp.stack([src, src + N_NODES]).reshape(NC, NS, NB1, IB, CHW1)
    dst1 = dst.reshape(NS, NB1, IB, CHW1)
    src2 = src.reshape(NC * NS, NB2, IB, CHW2)
    dst2 = dst.reshape(NC * NS, NB2, IB, CHW2)
    zeros1 = jnp.zeros((DRCH, 128), jnp.float32)
    zeros2 = jnp.zeros((DRCH, NCLS_P), jnp.float32)
    b1a = jnp.broadcast_to(b1[:128].reshape(1, 128), (8, 128))
    b1b = jnp.broadcast_to(b1[128:].reshape(1, 128), (8, 128))
    b2r = jnp.broadcast_to(b2.reshape(1, NCLASS), (8, NCLASS))
    W2p = jnp.pad(W2, ((0, 0), (0, NCLS_P - NCLASS)))

    sup = _mm1(x, W1)
    h_all = _spmm1_kernel()(sup, src1, dst1, zeros1)
    s2 = _mm2(h_all, b1a, b1b, W2p)
    p_all = _spmm2_kernel()(s2, src2, dst2, zeros2)
    return _comb(p_all, b2r)


# speedup vs baseline: 8.8253x; 1.0417x over previous
import functools

import jax
import jax.numpy as jnp
from jax import lax
from jax.experimental import pallas as pl
from jax.experimental.pallas import tpu as pltpu
from jax.experimental.pallas import tpu_sc as plsc

N_NODES = 10000
N_EDGES = 160000
NFEAT = 256
NHID = 256
NCLASS = 64
NCLS_P = 128

NC = 2
NS = 16

ROW_BLK = 1000
N_ROW_BLK = N_NODES // ROW_BLK

DRCH = 80
NDRCH = N_NODES // DRCH
DR_PER_TILE = -(-NDRCH // NS)

CHW1 = 80
CHW2 = 100
IB = 25
NB1 = N_EDGES // NS // (IB * CHW1)
NB2 = N_EDGES // (NC * NS) // (IB * CHW2)
D1 = 4
D2 = 4


@functools.lru_cache(maxsize=None)
def _mesh():
    return plsc.VectorSubcoreMesh(
        core_axis_name="c", subcore_axis_name="s", num_cores=NC, num_subcores=NS
    )


def _mm1_body(x_ref, w_ref, o_ref):
    o_ref[...] = jnp.dot(x_ref[...], w_ref[...],
                         preferred_element_type=jnp.float32)


def _mm1(x, W1):
    return pl.pallas_call(
        _mm1_body,
        grid=(NC, N_ROW_BLK),
        in_specs=[
            pl.BlockSpec((ROW_BLK, NFEAT), lambda h, i: (i, 0)),
            pl.BlockSpec((NFEAT, 128), lambda h, i: (0, h)),
        ],
        out_specs=pl.BlockSpec((ROW_BLK, 128),
                               lambda h, i: (h * N_ROW_BLK + i, 0)),
        out_shape=jax.ShapeDtypeStruct((NC * N_NODES, 128), jnp.float32),
    )(x, W1)


def _zero_acc(zeros_hbm, buf_v, acc, s):
    pltpu.sync_copy(zeros_hbm, buf_v)
    for i in range(DR_PER_TILE):
        ch = s + NS * i

        @pl.when(ch < NDRCH)
        def _():
            pltpu.sync_copy(buf_v, acc.at[pl.ds(ch * DRCH, DRCH)])


@functools.lru_cache(maxsize=None)
def _spmm1_kernel():
    @functools.partial(
        pl.kernel,
        out_type=jax.ShapeDtypeStruct((NC * N_NODES, 128), jnp.float32),
        mesh=_mesh(),
        scratch_types=[
            pltpu.VMEM((IB, CHW1), jnp.int32),
            pltpu.VMEM((IB, CHW1), jnp.int32),
            [pltpu.VMEM((CHW1, 128), jnp.float32) for _ in range(D1)],
            pltpu.VMEM_SHARED((N_NODES, 128), jnp.float32),
            [pltpu.SemaphoreType.DMA for _ in range(D1)],
        ],
    )
    def spmm1(sup_hbm, src_hbm, dst_hbm, zeros_hbm, h_hbm,
              isrc_v, idst_v, bufs, acc, sems):
        c = lax.axis_index("c")
        s = lax.axis_index("s")

        _zero_acc(zeros_hbm, bufs[0], acc, s)
        plsc.subcore_barrier()

        def gather(j, k):
            pltpu.async_copy(sup_hbm.at[isrc_v.at[j]], bufs[k], sems[k])

        def wait_gather(k):
            pltpu.make_async_copy(
                sup_hbm.at[isrc_v.at[0]], bufs[k], sems[k]).wait()

        def block_body(b, carry):
            pltpu.sync_copy(src_hbm.at[c, s, b], isrc_v)
            pltpu.sync_copy(dst_hbm.at[s, b], idst_v)
            for k in range(D1 - 1):
                gather(k, k)
            for j in range(IB):
                wait_gather(j % D1)
                if j + D1 - 1 < IB:
                    gather(j + D1 - 1, (j + D1 - 1) % D1)
                pltpu.sync_copy(bufs[j % D1], acc.at[idst_v.at[j]], add=True)
            return carry

        lax.fori_loop(0, NB1, block_body, 0)

        plsc.subcore_barrier()

        for i in range(DR_PER_TILE):
            ch = s + NS * i

            @pl.when(ch < NDRCH)
            def _():
                r0 = ch * DRCH
                pltpu.sync_copy(acc.at[pl.ds(r0, DRCH)], bufs[0])
                pltpu.sync_copy(bufs[0],
                                h_hbm.at[pl.ds(c * N_NODES + r0, DRCH)])

    return spmm1


def _mm2_body(h0_ref, h1_ref, b1a_ref, b1b_ref, w2_ref, o_ref):
    h0 = jnp.maximum(h0_ref[...] + b1a_ref[0:1, :], 0.0)
    h1 = jnp.maximum(h1_ref[...] + b1b_ref[0:1, :], 0.0)
    a = jnp.dot(h0, w2_ref[:128, :], preferred_element_type=jnp.float32)
    b = jnp.dot(h1, w2_ref[128:, :], preferred_element_type=jnp.float32)
    o_ref[...] = a + b


NCLS = NCLASS


def _mm2(h_all, b1a, b1b, W2):
    return pl.pallas_call(
        _mm2_body,
        grid=(N_ROW_BLK,),
        in_specs=[
            pl.BlockSpec((ROW_BLK, 128), lambda i: (i, 0)),
            pl.BlockSpec((ROW_BLK, 128), lambda i: (N_ROW_BLK + i, 0)),
            pl.BlockSpec((8, 128), lambda i: (0, 0)),
            pl.BlockSpec((8, 128), lambda i: (0, 0)),
            pl.BlockSpec((NHID, NCLS), lambda i: (0, 0)),
        ],
        out_specs=pl.BlockSpec((ROW_BLK, NCLS), lambda i: (i, 0)),
        out_shape=jax.ShapeDtypeStruct((N_NODES, NCLS), jnp.float32),
    )(h_all, h_all, b1a, b1b, W2)


@functools.lru_cache(maxsize=None)
def _spmm2_kernel():
    @functools.partial(
        pl.kernel,
        out_type=jax.ShapeDtypeStruct((NC * N_NODES, NCLS), jnp.float32),
        mesh=_mesh(),
        scratch_types=[
            pltpu.VMEM((IB, CHW2), jnp.int32),
            pltpu.VMEM((IB, CHW2), jnp.int32),
            [pltpu.VMEM((CHW2, NCLS), jnp.float32) for _ in range(D2)],
            pltpu.VMEM_SHARED((N_NODES, NCLS), jnp.float32),
            [pltpu.SemaphoreType.DMA for _ in range(D2)],
        ],
        compiler_params=pltpu.CompilerParams(use_tc_tiling_on_sc=False),
    )
    def spmm2(s2_hbm, src_hbm, dst_hbm, zeros_hbm, p_hbm,
              isrc_v, idst_v, bufs, acc, sems):
        c = lax.axis_index("c")
        s = lax.axis_index("s")
        wid = c * NS + s

        _zero_acc(zeros_hbm, bufs[0].at[pl.ds(0, DRCH)], acc, s)
        plsc.subcore_barrier()

        def gather(j, k):
            pltpu.async_copy(s2_hbm.at[isrc_v.at[j]], bufs[k], sems[k])

        def wait_gather(k):
            pltpu.make_async_copy(
                s2_hbm.at[isrc_v.at[0]], bufs[k], sems[k]).wait()

        def block_body(b, carry):
            pltpu.sync_copy(src_hbm.at[wid, b], isrc_v)
            pltpu.sync_copy(dst_hbm.at[wid, b], idst_v)
            for k in range(D2 - 1):
                gather(k, k)
            for j in range(IB):
                wait_gather(j % D2)
                if j + D2 - 1 < IB:
                    gather(j + D2 - 1, (j + D2 - 1) % D2)
                pltpu.sync_copy(bufs[j % D2], acc.at[idst_v.at[j]], add=True)
            return carry

        lax.fori_loop(0, NB2, block_body, 0)

        plsc.subcore_barrier()

        for i in range(DR_PER_TILE):
            ch = s + NS * i

            @pl.when(ch < NDRCH)
            def _():
                r0 = ch * DRCH
                pltpu.sync_copy(acc.at[pl.ds(r0, DRCH)], bufs[0].at[pl.ds(0, DRCH)])
                pltpu.sync_copy(bufs[0].at[pl.ds(0, DRCH)],
                                p_hbm.at[pl.ds(c * N_NODES + r0, DRCH)])

    return spmm2


def _comb_body(p0_ref, p1_ref, b2_ref, o_ref):
    o_ref[...] = p0_ref[...] + p1_ref[...] + b2_ref[0:1, :]


def _comb(p_all, b2):
    return pl.pallas_call(
        _comb_body,
        grid=(N_ROW_BLK,),
        in_specs=[
            pl.BlockSpec((ROW_BLK, NCLS), lambda i: (i, 0)),
            pl.BlockSpec((ROW_BLK, NCLS), lambda i: (N_ROW_BLK + i, 0)),
            pl.BlockSpec((8, NCLASS), lambda i: (0, 0)),
        ],
        out_specs=pl.BlockSpec((ROW_BLK, NCLASS), lambda i: (i, 0)),
        out_shape=jax.ShapeDtypeStruct((N_NODES, NCLASS), jnp.float32),
    )(p_all, p_all, b2)


@jax.jit
def kernel(x, adj, W1, b1, W2, b2):
    src = adj[0].astype(jnp.int32)
    dst = adj[1].astype(jnp.int32)
    src1 = jnp.stack([src, src + N_NODES]).reshape(NC, NS, NB1, IB, CHW1)
    dst1 = dst.reshape(NS, NB1, IB, CHW1)
    src2 = src.reshape(NC * NS, NB2, IB, CHW2)
    dst2 = dst.reshape(NC * NS, NB2, IB, CHW2)
    zeros1 = jnp.zeros((DRCH, 128), jnp.float32)
    zeros2 = jnp.zeros((DRCH, NCLASS), jnp.float32)
    b1a = jnp.broadcast_to(b1[:128].reshape(1, 128), (8, 128))
    b1b = jnp.broadcast_to(b1[128:].reshape(1, 128), (8, 128))
    b2r = jnp.broadcast_to(b2.reshape(1, NCLASS), (8, NCLASS))

    sup = _mm1(x, W1)
    h_all = _spmm1_kernel()(sup, src1, dst1, zeros1)
    s2 = _mm2(h_all, b1a, b1b, W2)
    p_all = _spmm2_kernel()(s2, src2, dst2, zeros2)
    return _comb(p_all, b2r)


# speedup vs baseline: 8.8980x; 1.0082x over previous
import functools

import jax
import jax.numpy as jnp
from jax import lax
from jax.experimental import pallas as pl
from jax.experimental.pallas import tpu as pltpu
from jax.experimental.pallas import tpu_sc as plsc

N_NODES = 10000
N_EDGES = 160000
NFEAT = 256
NHID = 256
NCLASS = 64
NCLS_P = 128

NC = 2
NS = 16

ROW_BLK = 1000
N_ROW_BLK = N_NODES // ROW_BLK

DRCH = 80
NDRCH = N_NODES // DRCH
DR_PER_TILE = -(-NDRCH // NS)

CHW1 = 80
CHW2 = 100
IB = 25
NB1 = N_EDGES // NS // (IB * CHW1)
NB2 = N_EDGES // (NC * NS) // (IB * CHW2)
D1 = 4
D2 = 4


@functools.lru_cache(maxsize=None)
def _mesh():
    return plsc.VectorSubcoreMesh(
        core_axis_name="c", subcore_axis_name="s", num_cores=NC, num_subcores=NS
    )


def _mm1_body(x_ref, w_ref, o_ref):
    o_ref[...] = jnp.dot(x_ref[...], w_ref[...],
                         preferred_element_type=jnp.float32)


def _mm1(x, W1):
    return pl.pallas_call(
        _mm1_body,
        grid=(NC, N_ROW_BLK),
        in_specs=[
            pl.BlockSpec((ROW_BLK, NFEAT), lambda h, i: (i, 0)),
            pl.BlockSpec((NFEAT, 128), lambda h, i: (0, h)),
        ],
        out_specs=pl.BlockSpec((ROW_BLK, 128),
                               lambda h, i: (h * N_ROW_BLK + i, 0)),
        out_shape=jax.ShapeDtypeStruct((NC * N_NODES, 128), jnp.float32),
    )(x, W1)


def _zero_acc(zeros_hbm, buf_v, acc, s):
    pltpu.sync_copy(zeros_hbm, buf_v)
    for i in range(DR_PER_TILE):
        ch = s + NS * i

        @pl.when(ch < NDRCH)
        def _():
            pltpu.sync_copy(buf_v, acc.at[pl.ds(ch * DRCH, DRCH)])


@functools.lru_cache(maxsize=None)
def _spmm1_kernel():
    @functools.partial(
        pl.kernel,
        out_type=jax.ShapeDtypeStruct((NC * N_NODES, 128), jnp.float32),
        mesh=_mesh(),
        scratch_types=[
            pltpu.VMEM((IB, CHW1), jnp.int32),
            pltpu.VMEM((IB, CHW1), jnp.int32),
            [pltpu.VMEM((CHW1, 128), jnp.float32) for _ in range(D1)],
            pltpu.VMEM_SHARED((N_NODES, 128), jnp.float32),
            [pltpu.SemaphoreType.DMA for _ in range(D1)],
        ],
    )
    def spmm1(sup_hbm, src_hbm, dst_hbm, zeros_hbm, h_hbm,
              isrc_v, idst_v, bufs, acc, sems):
        c = lax.axis_index("c")
        s = lax.axis_index("s")

        _zero_acc(zeros_hbm, bufs[0], acc, s)
        plsc.subcore_barrier()

        half = sup_hbm.at[pl.ds(c * N_NODES, N_NODES)]

        def gather(j, k):
            pltpu.async_copy(half.at[isrc_v.at[j]], bufs[k], sems[k])

        def wait_gather(k):
            pltpu.make_async_copy(
                half.at[isrc_v.at[0]], bufs[k], sems[k]).wait()

        def block_body(b, carry):
            pltpu.sync_copy(src_hbm.at[s, b], isrc_v)
            pltpu.sync_copy(dst_hbm.at[s, b], idst_v)
            for k in range(D1 - 1):
                gather(k, k)
            for j in range(IB):
                wait_gather(j % D1)
                if j + D1 - 1 < IB:
                    gather(j + D1 - 1, (j + D1 - 1) % D1)
                pltpu.sync_copy(bufs[j % D1], acc.at[idst_v.at[j]], add=True)
            return carry

        lax.fori_loop(0, NB1, block_body, 0)

        plsc.subcore_barrier()

        for i in range(DR_PER_TILE):
            ch = s + NS * i

            @pl.when(ch < NDRCH)
            def _():
                r0 = ch * DRCH
                pltpu.sync_copy(acc.at[pl.ds(r0, DRCH)], bufs[0])
                pltpu.sync_copy(bufs[0],
                                h_hbm.at[pl.ds(c * N_NODES + r0, DRCH)])

    return spmm1


def _mm2_body(h0_ref, h1_ref, b1a_ref, b1b_ref, w2_ref, o_ref):
    h0 = jnp.maximum(h0_ref[...] + b1a_ref[0:1, :], 0.0)
    h1 = jnp.maximum(h1_ref[...] + b1b_ref[0:1, :], 0.0)
    a = jnp.dot(h0, w2_ref[:128, :], preferred_element_type=jnp.float32)
    b = jnp.dot(h1, w2_ref[128:, :], preferred_element_type=jnp.float32)
    o_ref[...] = a + b


NCLS = NCLASS


def _mm2(h_all, b1a, b1b, W2):
    return pl.pallas_call(
        _mm2_body,
        grid=(N_ROW_BLK,),
        in_specs=[
            pl.BlockSpec((ROW_BLK, 128), lambda i: (i, 0)),
            pl.BlockSpec((ROW_BLK, 128), lambda i: (N_ROW_BLK + i, 0)),
            pl.BlockSpec((8, 128), lambda i: (0, 0)),
            pl.BlockSpec((8, 128), lambda i: (0, 0)),
            pl.BlockSpec((NHID, NCLS), lambda i: (0, 0)),
        ],
        out_specs=pl.BlockSpec((ROW_BLK, NCLS), lambda i: (i, 0)),
        out_shape=jax.ShapeDtypeStruct((N_NODES, NCLS), jnp.float32),
    )(h_all, h_all, b1a, b1b, W2)


@functools.lru_cache(maxsize=None)
def _spmm2_kernel():
    @functools.partial(
        pl.kernel,
        out_type=jax.ShapeDtypeStruct((NC * N_NODES, NCLS), jnp.float32),
        mesh=_mesh(),
        scratch_types=[
            pltpu.VMEM((IB, CHW2), jnp.int32),
            pltpu.VMEM((IB, CHW2), jnp.int32),
            [pltpu.VMEM((CHW2, NCLS), jnp.float32) for _ in range(D2)],
            pltpu.VMEM_SHARED((N_NODES, NCLS), jnp.float32),
            [pltpu.SemaphoreType.DMA for _ in range(D2)],
        ],
        compiler_params=pltpu.CompilerParams(use_tc_tiling_on_sc=False),
    )
    def spmm2(s2_hbm, src_hbm, dst_hbm, zeros_hbm, p_hbm,
              isrc_v, idst_v, bufs, acc, sems):
        c = lax.axis_index("c")
        s = lax.axis_index("s")
        wid = c * NS + s

        _zero_acc(zeros_hbm, bufs[0].at[pl.ds(0, DRCH)], acc, s)
        plsc.subcore_barrier()

        def gather(j, k):
            pltpu.async_copy(s2_hbm.at[isrc_v.at[j]], bufs[k], sems[k])

        def wait_gather(k):
            pltpu.make_async_copy(
                s2_hbm.at[isrc_v.at[0]], bufs[k], sems[k]).wait()

        def block_body(b, carry):
            pltpu.sync_copy(src_hbm.at[wid, b], isrc_v)
            pltpu.sync_copy(dst_hbm.at[wid, b], idst_v)
            for k in range(D2 - 1):
                gather(k, k)
            for j in range(IB):
                wait_gather(j % D2)
                if j + D2 - 1 < IB:
                    gather(j + D2 - 1, (j + D2 - 1) % D2)
                pltpu.sync_copy(bufs[j % D2], acc.at[idst_v.at[j]], add=True)
            return carry

        lax.fori_loop(0, NB2, block_body, 0)

        plsc.subcore_barrier()

        for i in range(DR_PER_TILE):
            ch = s + NS * i

            @pl.when(ch < NDRCH)
            def _():
                r0 = ch * DRCH
                pltpu.sync_copy(acc.at[pl.ds(r0, DRCH)], bufs[0].at[pl.ds(0, DRCH)])
                pltpu.sync_copy(bufs[0].at[pl.ds(0, DRCH)],
                                p_hbm.at[pl.ds(c * N_NODES + r0, DRCH)])

    return spmm2


def _comb_body(p0_ref, p1_ref, b2_ref, o_ref):
    o_ref[...] = p0_ref[...] + p1_ref[...] + b2_ref[0:1, :]


def _comb(p_all, b2):
    return pl.pallas_call(
        _comb_body,
        grid=(N_ROW_BLK,),
        in_specs=[
            pl.BlockSpec((ROW_BLK, NCLS), lambda i: (i, 0)),
            pl.BlockSpec((ROW_BLK, NCLS), lambda i: (N_ROW_BLK + i, 0)),
            pl.BlockSpec((8, NCLASS), lambda i: (0, 0)),
        ],
        out_specs=pl.BlockSpec((ROW_BLK, NCLASS), lambda i: (i, 0)),
        out_shape=jax.ShapeDtypeStruct((N_NODES, NCLASS), jnp.float32),
    )(p_all, p_all, b2)


@jax.jit
def kernel(x, adj, W1, b1, W2, b2):
    src = adj[0].astype(jnp.int32)
    dst = adj[1].astype(jnp.int32)
    src1 = src.reshape(NS, NB1, IB, CHW1)
    dst1 = dst.reshape(NS, NB1, IB, CHW1)
    src2 = src.reshape(NC * NS, NB2, IB, CHW2)
    dst2 = dst.reshape(NC * NS, NB2, IB, CHW2)
    zeros1 = jnp.zeros((DRCH, 128), jnp.float32)
    zeros2 = jnp.zeros((DRCH, NCLASS), jnp.float32)
    b1a = jnp.broadcast_to(b1[:128].reshape(1, 128), (8, 128))
    b1b = jnp.broadcast_to(b1[128:].reshape(1, 128), (8, 128))
    b2r = jnp.broadcast_to(b2.reshape(1, NCLASS), (8, NCLASS))

    sup = _mm1(x, W1)
    h_all = _spmm1_kernel()(sup, src1, dst1, zeros1)
    s2 = _mm2(h_all, b1a, b1b, W2)
    p_all = _spmm2_kernel()(s2, src2, dst2, zeros2)
    return _comb(p_all, b2r)
